# baseline (speedup 1.0000x reference)
import functools

import numpy as np
import jax
import jax.numpy as jnp
from jax import lax
from jax.experimental import pallas as pl
from jax.experimental.pallas import tpu as pltpu
from jax.experimental.pallas import tpu_sc as plsc

VOCAB = 1000000
B = 16384
D = 64
NG = 5
ROWS = 2 + NG
NC = 2
NS = 16
NW = NC * NS
L = 16
NPW = B // NW
M = 128
NCH = NPW * ROWS // M


def _dst_table() -> np.ndarray:
    dst = np.empty((NW, NCH, M), dtype=np.int32)
    for w in range(NW):
        base = w * NPW
        k = np.arange(NPW)
        a = (base + k) * ROWS
        b = a + 1
        kk = np.arange(NPW * NG)
        c = (base + kk // NG) * ROWS + 2 + kk % NG
        dst[w] = np.concatenate([a, b, c]).reshape(NCH, M)
    return dst


_DST_NP = _dst_table()


def _skipgram_gather(center, context, ng_words, dst_h, in_t2, out_t2):
    mesh = plsc.VectorSubcoreMesh(core_axis_name="c", subcore_axis_name="s")

    @functools.partial(
        pl.kernel,
        out_type=jax.ShapeDtypeStruct((B * ROWS, 2 * D), jnp.float32),
        mesh=mesh,
        scratch_types=[
            pltpu.VMEM((NPW * ROWS,), jnp.int32),
            pltpu.VMEM((NPW * ROWS,), jnp.int32),
            pltpu.VMEM((NCH, M), jnp.int32),
            pltpu.VMEM((M, 2 * D), jnp.float32),
            pltpu.VMEM((M, 2 * D), jnp.float32),
            pltpu.VMEM((M, 2 * D), jnp.float32),
            pltpu.VMEM((M, 2 * D), jnp.float32),
            pltpu.SemaphoreType.DMA,
            pltpu.SemaphoreType.DMA,
        ],
        compiler_params=pltpu.CompilerParams(use_tc_tiling_on_sc=True,
                                             needs_layout_passes=False),
    )
    def k(center_h, context_h, ng_h, dst_hbm, in_t, out_t, out_h,
          src_idx, pidx, dst_l, pbuf0, pbuf1, obuf0, obuf1, gsem, ssem):
        wid = lax.axis_index("s") * NC + lax.axis_index("c")
        base = wid * NPW

        pltpu.sync_copy(center_h.at[pl.ds(base, NPW)], src_idx.at[pl.ds(0, NPW)])
        pltpu.sync_copy(context_h.at[pl.ds(base, NPW)], src_idx.at[pl.ds(NPW, NPW)])
        pltpu.sync_copy(ng_h.at[pl.ds(base * NG, NPW * NG)],
                        src_idx.at[pl.ds(2 * NPW, NPW * NG)])
        pltpu.sync_copy(dst_hbm.at[wid], dst_l)

        def mk_pidx(i, carry):
            v = src_idx[pl.ds(i * L, L)]
            pidx[pl.ds(i * L, L)] = v >> 1
            return carry
        lax.fori_loop(0, NPW * ROWS // L, mk_pidx, 0)

        lane = lax.iota(jnp.int32, L)

        def run_phase(table, c0, nchunks):
            def gather(c, buf):
                src = table.at[pidx.at[pl.ds((c0 + c) * M, M)]]
                return pltpu.make_async_copy(src, buf, gsem)

            def scatter(c, buf):
                return pltpu.make_async_copy(buf, out_h.at[dst_l.at[c0 + c]], ssem)

            def extract(c, pbuf, obuf):
                off = (c0 + c) * M
                zero = lane * 0
                for g in range(M // L):
                    rv = src_idx[pl.ds(off + g * L, L)]
                    hv = (rv & 1) * D
                    rowpos = g * L + lane

                    def wbody(w, carry, hv=hv, rowpos=rowpos):
                        w8 = w * 8
                        for u in range(8):
                            vals = plsc.load_gather(pbuf, [rowpos, hv + (w8 + u)])
                            plsc.store_scatter(obuf, [rowpos, zero + (w8 + u)], vals)
                        return carry

                    lax.fori_loop(0, D // 8, wbody, 0)

            half = nchunks // 2
            gather(0, pbuf0).start()

            def body(i, carry):
                a = 2 * i

                gather(a, pbuf0).wait()
                gather(a + 1, pbuf1).start()
                extract(a, pbuf0, obuf0)

                @pl.when(i > 0)
                def _():
                    scatter(a - 2, obuf0).wait()

                scatter(a, obuf0).start()

                gather(a + 1, pbuf1).wait()

                @pl.when(i < half - 1)
                def _():
                    gather(a + 2, pbuf0).start()

                extract(a + 1, pbuf1, obuf1)

                @pl.when(i > 0)
                def _():
                    scatter(a - 1, obuf1).wait()

                scatter(a + 1, obuf1).start()
                return carry

            lax.fori_loop(0, half, body, 0)
            scatter(nchunks - 2, obuf0).wait()
            scatter(nchunks - 1, obuf1).wait()

        run_phase(in_t, 0, NPW // M)
        run_phase(out_t, NPW // M, NPW // M)
        run_phase(out_t, 2 * NPW // M, NPW * NG // M)

    return k(center, context, ng_words, dst_h, in_t2, out_t2)


@jax.jit
def kernel(center, context, in_table, out_table, ng_words):
    out = _skipgram_gather(center, context, ng_words, jnp.asarray(_DST_NP),
                           in_table.reshape(VOCAB // 2, 2 * D),
                           out_table.reshape(VOCAB // 2, 2 * D))
    return out[:, :D].reshape(B, ROWS, D)
